# baseline (speedup 1.0000x reference)
import jax
import jax.numpy as jnp
from jax.experimental import pallas as pl

_NUM_E = 8192
_DIM = 32
_N = 8192
_ROWS = 256
_STEPS = _N // _ROWS
_WIN = 2048


def _bf16_round(v):
    u = jax.lax.bitcast_convert_type(v, jnp.uint32)
    r = (u + jnp.uint32(0x7FFF) + ((u >> 16) & jnp.uint32(1))) & jnp.uint32(0xFFFF0000)
    return jax.lax.bitcast_convert_type(r, jnp.float32)


def _norms_body(xT_ref, eT_ref, fn_ref, cn_ref):
    xa = xT_ref[0:1, :] * xT_ref[0:1, :]
    ea = eT_ref[0:1, :] * eT_ref[0:1, :]
    for c in range(1, _DIM):
        xa = xa + xT_ref[c:c + 1, :] * xT_ref[c:c + 1, :]
        ea = ea + eT_ref[c:c + 1, :] * eT_ref[c:c + 1, :]
    fn_ref[...] = xa
    cn_ref[...] = ea


def _vq_body(x_ref, emb_ref, fn_ref, cn_ref, out_ref, idx_ref, loss_ref):
    x = x_ref[...]
    emb = emb_ref[...]
    fn = fn_ref[...]
    cn = cn_ref[...]
    xr = _bf16_round(x)
    mm = jax.lax.dot_general(xr, emb, (((1,), (1,)), ((), ())),
                             preferred_element_type=jnp.float32,
                             precision=jax.lax.Precision.HIGHEST)
    dist = (fn + cn) - 2.0 * mm
    accv = jnp.full((_ROWS, 1), jnp.inf, jnp.float32)
    acci = jnp.zeros((_ROWS, 1), jnp.int32)
    iota_w = jax.lax.broadcasted_iota(jnp.int32, (_ROWS, _WIN), 1)
    for c in range(_NUM_E // _WIN):
        blk = jax.lax.slice(dist, (0, c * _WIN), (_ROWS, (c + 1) * _WIN))
        bv = jnp.min(blk, axis=1, keepdims=True)
        bi = jnp.min(jnp.where(blk == bv, iota_w, _NUM_E), axis=1,
                     keepdims=True).astype(jnp.int32) + jnp.int32(c * _WIN)
        upd = bv < accv
        acci = jnp.where(upd, bi, acci)
        accv = _bf16_round(jnp.where(upd, bv, accv))
    idx_ref[...] = acci
    iota = jax.lax.broadcasted_iota(jnp.int32, (_ROWS, _NUM_E), 1)
    onehot = (iota == acci).astype(jnp.float32)
    q = jax.lax.dot_general(onehot, emb, (((1,), (0,)), ((), ())),
                            preferred_element_type=jnp.float32,
                            precision=jax.lax.Precision.HIGHEST)
    out_ref[...] = x + (q - x)
    step = pl.program_id(0)

    @pl.when(step == 0)
    def _init():
        loss_ref[...] = jnp.zeros_like(loss_ref)

    loss_ref[...] += jnp.sum((q - x) ** 2)


def kernel(inputs, embedding):
    B, C, H, W = inputs.shape
    flat = inputs.transpose(0, 2, 3, 1).reshape(_N, _DIM)
    fn, cn = pl.pallas_call(
        _norms_body,
        in_specs=[pl.BlockSpec((_DIM, _N), lambda: (0, 0)),
                  pl.BlockSpec((_DIM, _NUM_E), lambda: (0, 0))],
        out_specs=[pl.BlockSpec((1, _N), lambda: (0, 0)),
                   pl.BlockSpec((1, _NUM_E), lambda: (0, 0))],
        out_shape=[jax.ShapeDtypeStruct((1, _N), jnp.float32),
                   jax.ShapeDtypeStruct((1, _NUM_E), jnp.float32)],
    )(flat.T, embedding.T)
    out, idx, loss = pl.pallas_call(
        _vq_body,
        grid=(_STEPS,),
        in_specs=[
            pl.BlockSpec((_ROWS, _DIM), lambda i: (i, 0)),
            pl.BlockSpec((_NUM_E, _DIM), lambda i: (0, 0)),
            pl.BlockSpec((_ROWS, 1), lambda i: (i, 0)),
            pl.BlockSpec((1, _NUM_E), lambda i: (0, 0)),
        ],
        out_specs=[
            pl.BlockSpec((_ROWS, _DIM), lambda i: (i, 0)),
            pl.BlockSpec((_ROWS, 1), lambda i: (i, 0)),
            pl.BlockSpec((1, 1), lambda i: (0, 0)),
        ],
        out_shape=[
            jax.ShapeDtypeStruct((_N, _DIM), jnp.float32),
            jax.ShapeDtypeStruct((_N, 1), jnp.int32),
            jax.ShapeDtypeStruct((1, 1), jnp.float32),
        ],
    )(flat, embedding, fn.reshape(_N, 1), cn)
    quantized_out = out.reshape(B, H, W, C).transpose(0, 3, 1, 2)
    vq_loss = loss[0, 0] * (0.25 / (_N * _DIM))
    return (quantized_out, vq_loss, idx.reshape(_N))
